# Initial kernel scaffold; baseline (speedup 1.0000x reference)
import jax
import jax.numpy as jnp
from jax.experimental import pallas as pl


def kernel(overlap, scene, embed_table):
    raise NotImplementedError("write your pallas kernel here")


# speedup vs baseline: 3.1334x; 3.1334x over previous
import jax
import jax.numpy as jnp
from jax import lax
from jax.experimental import pallas as pl
from jax.experimental.pallas import tpu as pltpu
from jax.experimental.pallas import tpu_sc as plsc

EMBED_DIM = 128
BATCH = 16384
NUM_CORES = 2
NUM_SUBCORES = 16
NUM_WORKERS = NUM_CORES * NUM_SUBCORES
BPW = BATCH // NUM_WORKERS
CHUNK = 128
NCHUNK = BPW // CHUNK


def _gather_body(table_hbm, ov_hbm, sc_hbm, out_ov, out_sc,
                 idx_a, rows_a, idx_b, rows_b, sem_a, sem_b):
    wid = lax.axis_index("s") * NUM_CORES + lax.axis_index("c")
    base = wid * BPW

    idx_bufs = (idx_a, idx_b)
    row_bufs = (rows_a, rows_b)
    sems = (sem_a, sem_b)

    def chunk_src_dst(k):
        if k < NCHUNK:
            return ov_hbm, out_ov, base + k * CHUNK
        return sc_hbm, out_sc, base + (k - NCHUNK) * CHUNK

    total = 2 * NCHUNK
    copies = [None, None]
    for k in range(total):
        slot = k % 2
        src, dst, off = chunk_src_dst(k)
        pltpu.sync_copy(src.at[pl.ds(off, CHUNK)], idx_bufs[slot])
        copies[slot] = pltpu.async_copy(
            table_hbm.at[idx_bufs[slot]], row_bufs[slot], sems[slot])
        if k >= 1:
            pslot = (k - 1) % 2
            _, pdst, poff = chunk_src_dst(k - 1)
            copies[pslot].wait()
            pltpu.sync_copy(row_bufs[pslot], pdst.at[pl.ds(poff, CHUNK)])
    lslot = (total - 1) % 2
    _, ldst, loff = chunk_src_dst(total - 1)
    copies[lslot].wait()
    pltpu.sync_copy(row_bufs[lslot], ldst.at[pl.ds(loff, CHUNK)])


@jax.jit
def kernel(overlap, scene, embed_table):
    ov = overlap.astype(jnp.int32)
    sc = scene.astype(jnp.int32)
    out_sds = jax.ShapeDtypeStruct((BATCH, EMBED_DIM), jnp.float32)
    run = pl.kernel(
        _gather_body,
        out_type=(out_sds, out_sds),
        mesh=plsc.VectorSubcoreMesh(core_axis_name="c", subcore_axis_name="s"),
        scratch_types=[
            pltpu.VMEM((CHUNK,), jnp.int32),
            pltpu.VMEM((CHUNK, EMBED_DIM), jnp.float32),
            pltpu.VMEM((CHUNK,), jnp.int32),
            pltpu.VMEM((CHUNK, EMBED_DIM), jnp.float32),
            pltpu.SemaphoreType.DMA,
            pltpu.SemaphoreType.DMA,
        ],
    )
    return run(embed_table, ov, sc)
